# baseline (speedup 1.0000x reference)
import jax
import jax.numpy as jnp
from jax.experimental import pallas as pl
from jax.experimental.pallas import tpu as pltpu
from jax.experimental.pallas import tpu_sc as plsc

_S = 28
_B = 3
_NUM_CLASSES = 80
_C = _B * 5 + _NUM_CLASSES
_LAMDA_COORD = 5.0
_LAMDA_NOOBJ = 0.5


def _dense_body(sizes_ref, mo_ref, tg_ref, out_ref):
    i = pl.program_id(0)
    mo = mo_ref[0]
    tg = tg_ref[0]

    conf_t = tg[..., _B * 4:_B * 5]
    pred_conf = mo[..., _B * 4:_B * 5]
    noobj_mask = jnp.sum(conf_t, axis=2) == 0.0
    noobj_loss = _LAMDA_NOOBJ * jnp.sum(
        jnp.where(noobj_mask[..., None], pred_conf, 0.0) ** 2)

    cnt = jnp.sum((conf_t == 1.0).astype(jnp.float32), axis=2)
    has_obj = cnt > 0

    H_ = sizes_ref[i, 0]
    W_ = sizes_ref[i, 1]
    gh = H_ / _S
    gw = W_ / _S
    xg = jax.lax.broadcasted_iota(jnp.int32, (_S, _S), 1).astype(jnp.float32)
    yg = jax.lax.broadcasted_iota(jnp.int32, (_S, _S), 0).astype(jnp.float32)
    x_base = xg * gw
    y_base = yg * gh

    tx = tg[..., 0] * gw + x_base
    ty = tg[..., 1] * gh + y_base
    tw = tg[..., _B * 2] * W_
    th = tg[..., _B * 2 + 1] * H_
    tx1 = tx - tw / 2
    ty1 = ty - th / 2
    tx2 = tx + tw / 2
    ty2 = ty + th / 2
    area_t = (tx2 - tx1) * (ty2 - ty1)

    best = jnp.zeros((_S, _S), jnp.int32)
    best_iou = jnp.full((_S, _S), -jnp.inf)
    ious = []
    for b in range(_B):
        px = mo[..., 2 * b] * gw + x_base
        py = mo[..., 2 * b + 1] * gh + y_base
        pw = mo[..., _B * 2 + 2 * b] * W_
        ph = mo[..., _B * 2 + 2 * b + 1] * H_
        px1 = px - pw / 2
        py1 = py - ph / 2
        px2 = px + pw / 2
        py2 = py + ph / 2
        area_p = (px2 - px1) * (py2 - py1)
        lx = jnp.maximum(tx1, px1)
        ly = jnp.maximum(ty1, py1)
        rx = jnp.minimum(tx2, px2)
        ry = jnp.minimum(ty2, py2)
        iw = jnp.clip(rx - lx, 0.0)
        ih = jnp.clip(ry - ly, 0.0)
        inter = iw * ih
        denom = area_t + area_p - inter
        iou = inter / jnp.where(has_obj, denom, 1.0)
        ious.append(iou)
        take = iou > best_iou
        best = jnp.where(take, b, best)
        best_iou = jnp.where(take, iou, best_iou)

    p_x_raw = jnp.zeros((_S, _S), jnp.float32)
    p_y_raw = jnp.zeros((_S, _S), jnp.float32)
    p_w_raw = jnp.zeros((_S, _S), jnp.float32)
    p_h_raw = jnp.zeros((_S, _S), jnp.float32)
    p_conf_resp = jnp.zeros((_S, _S), jnp.float32)
    noobj_cell = jnp.zeros((_S, _S), jnp.float32)
    for b in range(_B):
        is_b = best == b
        p_x_raw = jnp.where(is_b, mo[..., 2 * b], p_x_raw)
        p_y_raw = jnp.where(is_b, mo[..., 2 * b + 1], p_y_raw)
        p_w_raw = jnp.where(is_b, mo[..., _B * 2 + 2 * b], p_w_raw)
        p_h_raw = jnp.where(is_b, mo[..., _B * 2 + 2 * b + 1], p_h_raw)
        pc_b = mo[..., _B * 4 + b]
        p_conf_resp = jnp.where(is_b, pc_b, p_conf_resp)
        noobj_cell = noobj_cell + jnp.where(is_b, 0.0, pc_b) ** 2

    t_x_raw = tg[..., 0]
    t_y_raw = tg[..., 1]
    t_w_raw = tg[..., _B * 2]
    t_h_raw = tg[..., _B * 2 + 1]
    coord_cell = ((p_x_raw - t_x_raw) ** 2 + (p_y_raw - t_y_raw) ** 2
                  + (jnp.sqrt(jnp.clip(p_w_raw, 0.0)) - jnp.sqrt(jnp.clip(t_w_raw, 0.0))) ** 2
                  + (jnp.sqrt(jnp.clip(p_h_raw, 0.0)) - jnp.sqrt(jnp.clip(t_h_raw, 0.0))) ** 2)
    conf_cell = (p_conf_resp - best_iou) ** 2
    cls_cell = jnp.sum((mo[..., -_NUM_CLASSES:] - tg[..., -_NUM_CLASSES:]) ** 2, axis=-1)
    per_cell = (_LAMDA_COORD * coord_cell + conf_cell
                + _LAMDA_NOOBJ * noobj_cell + cls_cell)
    obj_loss = jnp.sum(jnp.where(has_obj, cnt * per_cell, 0.0))

    blk = noobj_loss + obj_loss

    @pl.when(pl.program_id(0) == 0)
    def _():
        out_ref[0, 0] = 0.0

    out_ref[0, 0] += blk


def _dense_loss(model_output, targets, orig_image_sizes):
    n = model_output.shape[0]
    grid = (n,)
    out = pl.pallas_call(
        _dense_body,
        grid=grid,
        in_specs=[
            pl.BlockSpec(memory_space=pltpu.SMEM),
            pl.BlockSpec((1, _S, _S, _C), lambda i: (i, 0, 0, 0)),
            pl.BlockSpec((1, _S, _S, _C), lambda i: (i, 0, 0, 0)),
        ],
        out_specs=pl.BlockSpec(memory_space=pltpu.SMEM),
        out_shape=jax.ShapeDtypeStruct((1, 1), jnp.float32),
    )(orig_image_sizes, model_output, targets)
    return out[0, 0]


_NW = 32
_CELLS = 512 * _S * _S
_CPW = _CELLS // _NW


_NCHUNK = 8
_CH = _CPW // _NCHUNK


def _sc_conf_kernel(mo_hbm, tg_hbm, out_hbm, mo_buf, tg_buf, stage):
    import jax.lax as lax
    wid = lax.axis_index("s") * 2 + lax.axis_index("c")
    c0 = wid * _CPW

    iota16 = lax.broadcasted_iota(jnp.int32, (16,), 0)
    zeros_i = jnp.full((16,), _B * 4 - 8, jnp.int32)
    ones_i = jnp.full((16,), _B * 4 - 7, jnp.int32)
    twos_i = jnp.full((16,), _B * 4 - 6, jnp.int32)

    def body(i, carry):
        nl, tmax = carry
        ci = i * 16 + iota16
        t0 = plsc.load_gather(tg_buf, [ci, zeros_i])
        t1 = plsc.load_gather(tg_buf, [ci, ones_i])
        t2 = plsc.load_gather(tg_buf, [ci, twos_i])
        p0 = plsc.load_gather(mo_buf, [ci, zeros_i])
        p1 = plsc.load_gather(mo_buf, [ci, ones_i])
        p2 = plsc.load_gather(mo_buf, [ci, twos_i])
        s = t0 + t1 + t2
        m = s == 0.0
        sq = p0 * p0 + p1 * p1 + p2 * p2
        nl = nl + jnp.where(m, sq, 0.0)
        tmax = jnp.maximum(tmax, jnp.maximum(t0, jnp.maximum(t1, t2)))
        return nl, tmax

    nl = jnp.zeros((16,), jnp.float32)
    tmax = jnp.full((16,), -jnp.inf, jnp.float32)
    for k in range(_NCHUNK):
        ck = c0 + k * _CH
        pltpu.sync_copy(mo_hbm.at[pl.ds(ck, _CH), pl.ds(8, 8)], mo_buf)
        pltpu.sync_copy(tg_hbm.at[pl.ds(ck, _CH), pl.ds(8, 8)], tg_buf)
        nl, tmax = lax.fori_loop(0, _CH // 16, body, (nl, tmax))

    stage[...] = nl
    pltpu.sync_copy(stage, out_hbm.at[wid, pl.ds(0, 16)])
    stage[...] = tmax
    pltpu.sync_copy(stage, out_hbm.at[wid, pl.ds(16, 16)])


def _sc_conf_stats(model_output, targets):
    import functools
    mo_v = model_output.reshape(_CELLS, _C)
    tg_v = targets.reshape(_CELLS, _C)
    mesh = plsc.VectorSubcoreMesh(core_axis_name="c", subcore_axis_name="s")
    f = functools.partial(
        pl.kernel,
        mesh=mesh,
        out_type=jax.ShapeDtypeStruct((_NW, 32), jnp.float32),
        scratch_types=[
            pltpu.VMEM((_CH, 8), jnp.float32),
            pltpu.VMEM((_CH, 8), jnp.float32),
            pltpu.VMEM((16,), jnp.float32),
        ],
        compiler_params=pltpu.CompilerParams(
            use_tc_tiling_on_sc=False, needs_layout_passes=False),
    )(_sc_conf_kernel)
    return f(mo_v, tg_v)


def kernel(model_output, targets, orig_image_sizes):
    stats = _sc_conf_stats(model_output, targets)
    noobj_loss = _LAMDA_NOOBJ * jnp.sum(stats[:, :16])
    any_obj = jnp.max(stats[:, 16:]) >= 1.0
    return jax.lax.cond(
        any_obj,
        lambda: _dense_loss(model_output, targets, orig_image_sizes),
        lambda: noobj_loss,
    )


# speedup vs baseline: 26.6134x; 26.6134x over previous
import jax
import jax.numpy as jnp
from jax.experimental import pallas as pl
from jax.experimental.pallas import tpu as pltpu

_S = 28
_B = 3
_NUM_CLASSES = 80
_C = _B * 5 + _NUM_CLASSES
_LAMDA_COORD = 5.0
_LAMDA_NOOBJ = 0.5


def _dense_body(sizes_ref, mo_ref, tg_ref, out_ref):
    i = pl.program_id(0)
    mo = mo_ref[0]
    tg = tg_ref[0]

    conf_t = tg[..., _B * 4:_B * 5]
    pred_conf = mo[..., _B * 4:_B * 5]
    noobj_mask = jnp.sum(conf_t, axis=2) == 0.0
    noobj_loss = _LAMDA_NOOBJ * jnp.sum(
        jnp.where(noobj_mask[..., None], pred_conf, 0.0) ** 2)

    cnt = jnp.sum((conf_t == 1.0).astype(jnp.float32), axis=2)
    has_obj = cnt > 0

    H_ = sizes_ref[i, 0]
    W_ = sizes_ref[i, 1]
    gh = H_ / _S
    gw = W_ / _S
    xg = jax.lax.broadcasted_iota(jnp.int32, (_S, _S), 1).astype(jnp.float32)
    yg = jax.lax.broadcasted_iota(jnp.int32, (_S, _S), 0).astype(jnp.float32)
    x_base = xg * gw
    y_base = yg * gh

    tx = tg[..., 0] * gw + x_base
    ty = tg[..., 1] * gh + y_base
    tw = tg[..., _B * 2] * W_
    th = tg[..., _B * 2 + 1] * H_
    tx1 = tx - tw / 2
    ty1 = ty - th / 2
    tx2 = tx + tw / 2
    ty2 = ty + th / 2
    area_t = (tx2 - tx1) * (ty2 - ty1)

    best = jnp.zeros((_S, _S), jnp.int32)
    best_iou = jnp.full((_S, _S), -jnp.inf)
    ious = []
    for b in range(_B):
        px = mo[..., 2 * b] * gw + x_base
        py = mo[..., 2 * b + 1] * gh + y_base
        pw = mo[..., _B * 2 + 2 * b] * W_
        ph = mo[..., _B * 2 + 2 * b + 1] * H_
        px1 = px - pw / 2
        py1 = py - ph / 2
        px2 = px + pw / 2
        py2 = py + ph / 2
        area_p = (px2 - px1) * (py2 - py1)
        lx = jnp.maximum(tx1, px1)
        ly = jnp.maximum(ty1, py1)
        rx = jnp.minimum(tx2, px2)
        ry = jnp.minimum(ty2, py2)
        iw = jnp.clip(rx - lx, 0.0)
        ih = jnp.clip(ry - ly, 0.0)
        inter = iw * ih
        denom = area_t + area_p - inter
        iou = inter / jnp.where(has_obj, denom, 1.0)
        ious.append(iou)
        take = iou > best_iou
        best = jnp.where(take, b, best)
        best_iou = jnp.where(take, iou, best_iou)

    p_x_raw = jnp.zeros((_S, _S), jnp.float32)
    p_y_raw = jnp.zeros((_S, _S), jnp.float32)
    p_w_raw = jnp.zeros((_S, _S), jnp.float32)
    p_h_raw = jnp.zeros((_S, _S), jnp.float32)
    p_conf_resp = jnp.zeros((_S, _S), jnp.float32)
    noobj_cell = jnp.zeros((_S, _S), jnp.float32)
    for b in range(_B):
        is_b = best == b
        p_x_raw = jnp.where(is_b, mo[..., 2 * b], p_x_raw)
        p_y_raw = jnp.where(is_b, mo[..., 2 * b + 1], p_y_raw)
        p_w_raw = jnp.where(is_b, mo[..., _B * 2 + 2 * b], p_w_raw)
        p_h_raw = jnp.where(is_b, mo[..., _B * 2 + 2 * b + 1], p_h_raw)
        pc_b = mo[..., _B * 4 + b]
        p_conf_resp = jnp.where(is_b, pc_b, p_conf_resp)
        noobj_cell = noobj_cell + jnp.where(is_b, 0.0, pc_b) ** 2

    t_x_raw = tg[..., 0]
    t_y_raw = tg[..., 1]
    t_w_raw = tg[..., _B * 2]
    t_h_raw = tg[..., _B * 2 + 1]
    coord_cell = ((p_x_raw - t_x_raw) ** 2 + (p_y_raw - t_y_raw) ** 2
                  + (jnp.sqrt(jnp.clip(p_w_raw, 0.0)) - jnp.sqrt(jnp.clip(t_w_raw, 0.0))) ** 2
                  + (jnp.sqrt(jnp.clip(p_h_raw, 0.0)) - jnp.sqrt(jnp.clip(t_h_raw, 0.0))) ** 2)
    conf_cell = (p_conf_resp - best_iou) ** 2
    cls_cell = jnp.sum((mo[..., -_NUM_CLASSES:] - tg[..., -_NUM_CLASSES:]) ** 2, axis=-1)
    per_cell = (_LAMDA_COORD * coord_cell + conf_cell
                + _LAMDA_NOOBJ * noobj_cell + cls_cell)
    obj_loss = jnp.sum(jnp.where(has_obj, cnt * per_cell, 0.0))

    blk = noobj_loss + obj_loss

    @pl.when(pl.program_id(0) == 0)
    def _():
        out_ref[0, 0] = 0.0

    out_ref[0, 0] += blk


def _dense_loss(model_output, targets, orig_image_sizes):
    n = model_output.shape[0]
    grid = (n,)
    out = pl.pallas_call(
        _dense_body,
        grid=grid,
        in_specs=[
            pl.BlockSpec(memory_space=pltpu.SMEM),
            pl.BlockSpec((1, _S, _S, _C), lambda i: (i, 0, 0, 0)),
            pl.BlockSpec((1, _S, _S, _C), lambda i: (i, 0, 0, 0)),
        ],
        out_specs=pl.BlockSpec(memory_space=pltpu.SMEM),
        out_shape=jax.ShapeDtypeStruct((1, 1), jnp.float32),
    )(orig_image_sizes, model_output, targets)
    return out[0, 0]


def _conf_body(cp_ref, ct_ref, out_ref):
    cp = cp_ref[...]
    ct = ct_ref[...]
    w = cp.shape[1]
    start = (jax.lax.broadcasted_iota(jnp.int32, cp.shape, 1) % 3) == 0
    s = ct + jnp.roll(ct, -1, axis=1) + jnp.roll(ct, -2, axis=1)
    noobj = (s == 0.0) & start
    q = cp * cp
    qs = q + jnp.roll(q, -1, axis=1) + jnp.roll(q, -2, axis=1)
    nl = jnp.sum(jnp.where(noobj, qs, 0.0))
    cnt = jnp.sum((ct == 1.0).astype(jnp.float32))

    @pl.when(pl.program_id(0) == 0)
    def _():
        out_ref[0, 0] = 0.0
        out_ref[0, 1] = 0.0

    out_ref[0, 0] += nl
    out_ref[0, 1] += cnt


def kernel(model_output, targets, orig_image_sizes):
    n = model_output.shape[0]
    cp = model_output[..., _B * 4:_B * 5].reshape(n, _S * _S * _B)
    ct = targets[..., _B * 4:_B * 5].reshape(n, _S * _S * _B)
    nbi = 64
    out = pl.pallas_call(
        _conf_body,
        grid=(n // nbi,),
        in_specs=[
            pl.BlockSpec((nbi, _S * _S * _B), lambda i: (i, 0)),
            pl.BlockSpec((nbi, _S * _S * _B), lambda i: (i, 0)),
        ],
        out_specs=pl.BlockSpec(memory_space=pltpu.SMEM),
        out_shape=jax.ShapeDtypeStruct((1, 2), jnp.float32),
    )(cp, ct)
    noobj_loss = _LAMDA_NOOBJ * out[0, 0]
    any_obj = out[0, 1] > 0.0
    return jax.lax.cond(
        any_obj,
        lambda: _dense_loss(model_output, targets, orig_image_sizes),
        lambda: noobj_loss,
    )


# speedup vs baseline: 32.6556x; 1.2270x over previous
import jax
import jax.numpy as jnp
from jax.experimental import pallas as pl
from jax.experimental.pallas import tpu as pltpu

_S = 28
_B = 3
_NUM_CLASSES = 80
_C = _B * 5 + _NUM_CLASSES
_LAMDA_COORD = 5.0
_LAMDA_NOOBJ = 0.5


def _dense_body(sizes_ref, mo_ref, tg_ref, out_ref):
    i = pl.program_id(0)
    mo = mo_ref[0]
    tg = tg_ref[0]

    conf_t = tg[..., _B * 4:_B * 5]
    pred_conf = mo[..., _B * 4:_B * 5]
    noobj_mask = jnp.sum(conf_t, axis=2) == 0.0
    noobj_loss = _LAMDA_NOOBJ * jnp.sum(
        jnp.where(noobj_mask[..., None], pred_conf, 0.0) ** 2)

    cnt = jnp.sum((conf_t == 1.0).astype(jnp.float32), axis=2)
    has_obj = cnt > 0

    H_ = sizes_ref[i, 0]
    W_ = sizes_ref[i, 1]
    gh = H_ / _S
    gw = W_ / _S
    xg = jax.lax.broadcasted_iota(jnp.int32, (_S, _S), 1).astype(jnp.float32)
    yg = jax.lax.broadcasted_iota(jnp.int32, (_S, _S), 0).astype(jnp.float32)
    x_base = xg * gw
    y_base = yg * gh

    tx = tg[..., 0] * gw + x_base
    ty = tg[..., 1] * gh + y_base
    tw = tg[..., _B * 2] * W_
    th = tg[..., _B * 2 + 1] * H_
    tx1 = tx - tw / 2
    ty1 = ty - th / 2
    tx2 = tx + tw / 2
    ty2 = ty + th / 2
    area_t = (tx2 - tx1) * (ty2 - ty1)

    best = jnp.zeros((_S, _S), jnp.int32)
    best_iou = jnp.full((_S, _S), -jnp.inf)
    ious = []
    for b in range(_B):
        px = mo[..., 2 * b] * gw + x_base
        py = mo[..., 2 * b + 1] * gh + y_base
        pw = mo[..., _B * 2 + 2 * b] * W_
        ph = mo[..., _B * 2 + 2 * b + 1] * H_
        px1 = px - pw / 2
        py1 = py - ph / 2
        px2 = px + pw / 2
        py2 = py + ph / 2
        area_p = (px2 - px1) * (py2 - py1)
        lx = jnp.maximum(tx1, px1)
        ly = jnp.maximum(ty1, py1)
        rx = jnp.minimum(tx2, px2)
        ry = jnp.minimum(ty2, py2)
        iw = jnp.clip(rx - lx, 0.0)
        ih = jnp.clip(ry - ly, 0.0)
        inter = iw * ih
        denom = area_t + area_p - inter
        iou = inter / jnp.where(has_obj, denom, 1.0)
        ious.append(iou)
        take = iou > best_iou
        best = jnp.where(take, b, best)
        best_iou = jnp.where(take, iou, best_iou)

    p_x_raw = jnp.zeros((_S, _S), jnp.float32)
    p_y_raw = jnp.zeros((_S, _S), jnp.float32)
    p_w_raw = jnp.zeros((_S, _S), jnp.float32)
    p_h_raw = jnp.zeros((_S, _S), jnp.float32)
    p_conf_resp = jnp.zeros((_S, _S), jnp.float32)
    noobj_cell = jnp.zeros((_S, _S), jnp.float32)
    for b in range(_B):
        is_b = best == b
        p_x_raw = jnp.where(is_b, mo[..., 2 * b], p_x_raw)
        p_y_raw = jnp.where(is_b, mo[..., 2 * b + 1], p_y_raw)
        p_w_raw = jnp.where(is_b, mo[..., _B * 2 + 2 * b], p_w_raw)
        p_h_raw = jnp.where(is_b, mo[..., _B * 2 + 2 * b + 1], p_h_raw)
        pc_b = mo[..., _B * 4 + b]
        p_conf_resp = jnp.where(is_b, pc_b, p_conf_resp)
        noobj_cell = noobj_cell + jnp.where(is_b, 0.0, pc_b) ** 2

    t_x_raw = tg[..., 0]
    t_y_raw = tg[..., 1]
    t_w_raw = tg[..., _B * 2]
    t_h_raw = tg[..., _B * 2 + 1]
    coord_cell = ((p_x_raw - t_x_raw) ** 2 + (p_y_raw - t_y_raw) ** 2
                  + (jnp.sqrt(jnp.clip(p_w_raw, 0.0)) - jnp.sqrt(jnp.clip(t_w_raw, 0.0))) ** 2
                  + (jnp.sqrt(jnp.clip(p_h_raw, 0.0)) - jnp.sqrt(jnp.clip(t_h_raw, 0.0))) ** 2)
    conf_cell = (p_conf_resp - best_iou) ** 2
    cls_cell = jnp.sum((mo[..., -_NUM_CLASSES:] - tg[..., -_NUM_CLASSES:]) ** 2, axis=-1)
    per_cell = (_LAMDA_COORD * coord_cell + conf_cell
                + _LAMDA_NOOBJ * noobj_cell + cls_cell)
    obj_loss = jnp.sum(jnp.where(has_obj, cnt * per_cell, 0.0))

    blk = noobj_loss + obj_loss

    @pl.when(pl.program_id(0) == 0)
    def _():
        out_ref[0, 0] = 0.0

    out_ref[0, 0] += blk


def _dense_loss(model_output, targets, orig_image_sizes):
    n = model_output.shape[0]
    grid = (n,)
    out = pl.pallas_call(
        _dense_body,
        grid=grid,
        in_specs=[
            pl.BlockSpec(memory_space=pltpu.SMEM),
            pl.BlockSpec((1, _S, _S, _C), lambda i: (i, 0, 0, 0)),
            pl.BlockSpec((1, _S, _S, _C), lambda i: (i, 0, 0, 0)),
        ],
        out_specs=pl.BlockSpec(memory_space=pltpu.SMEM),
        out_shape=jax.ShapeDtypeStruct((1, 1), jnp.float32),
    )(orig_image_sizes, model_output, targets)
    return out[0, 0]


def _conf_body(cp_ref, ct_ref, out_ref):
    cp = cp_ref[...]
    ct = ct_ref[...]
    csum = ct[0] + ct[1] + ct[2]
    noobj = csum == 0.0
    nl = jnp.sum(jnp.where(noobj, cp[0], 0.0) ** 2
                 + jnp.where(noobj, cp[1], 0.0) ** 2
                 + jnp.where(noobj, cp[2], 0.0) ** 2)
    cnt = jnp.sum((ct == 1.0).astype(jnp.float32))

    @pl.when(pl.program_id(0) == 0)
    def _():
        out_ref[0, 0] = 0.0
        out_ref[0, 1] = 0.0

    out_ref[0, 0] += nl
    out_ref[0, 1] += cnt


def kernel(model_output, targets, orig_image_sizes):
    n = model_output.shape[0]
    cp = model_output[..., _B * 4:_B * 5].transpose(3, 0, 1, 2).reshape(_B, n, _S * _S)
    ct = targets[..., _B * 4:_B * 5].transpose(3, 0, 1, 2).reshape(_B, n, _S * _S)
    nbi = 256
    out = pl.pallas_call(
        _conf_body,
        grid=(n // nbi,),
        in_specs=[
            pl.BlockSpec((_B, nbi, _S * _S), lambda i: (0, i, 0)),
            pl.BlockSpec((_B, nbi, _S * _S), lambda i: (0, i, 0)),
        ],
        out_specs=pl.BlockSpec(memory_space=pltpu.SMEM),
        out_shape=jax.ShapeDtypeStruct((1, 2), jnp.float32),
    )(cp, ct)
    noobj_loss = _LAMDA_NOOBJ * out[0, 0]
    any_obj = out[0, 1] > 0.0
    return jax.lax.cond(
        any_obj,
        lambda: _dense_loss(model_output, targets, orig_image_sizes),
        lambda: noobj_loss,
    )
